# baseline (speedup 1.0000x reference)
import jax
import jax.numpy as jnp
from jax.experimental import pallas as pl
from jax.experimental.pallas import tpu as pltpu

V = 32000
PAD = 0
SMOOTHING = 0.1
CONF = 1.0 - SMOOTHING
EPS = SMOOTHING / (V - 2)

BR = 256
BC = 3200
NC = V // BC


def _loss_kernel(t_ref, x_ref, out_ref, m_ref, s_ref, w_ref):
    c = pl.program_id(1)
    x = x_ref[...]

    @pl.when(c == 0)
    def _init():
        m_ref[...] = jnp.full((BR, 1), -jnp.inf, jnp.float32)
        s_ref[...] = jnp.zeros((BR, 1), jnp.float32)
        w_ref[...] = jnp.zeros((BR, 1), jnp.float32)

    cmax = jnp.max(x, axis=1, keepdims=True)
    m_old = m_ref[...]
    m_new = jnp.maximum(m_old, cmax)
    alpha = jnp.exp(m_old - m_new)
    s_ref[...] = s_ref[...] * alpha + jnp.sum(
        jnp.exp(x - m_new), axis=1, keepdims=True)
    m_ref[...] = m_new

    t = t_ref[0, 0, :]
    col = jax.lax.broadcasted_iota(jnp.int32, (BR, BC), 1) + c * BC
    wt = jnp.where(col == t[:, None], CONF, EPS)
    wt = jnp.where(col == PAD, 0.0, wt)
    w_ref[...] = w_ref[...] + jnp.sum(wt * x, axis=1, keepdims=True)

    @pl.when(c == NC - 1)
    def _finish():
        L = m_ref[...] + jnp.log(s_ref[...])
        loss = L - w_ref[...]
        loss = jnp.where(t[:, None] == PAD, 0.0, loss)
        out_ref[...] = loss


def kernel(pred, target):
    n = pred.shape[0] * pred.shape[1]
    p = pred.reshape(n, V)
    t = target.reshape(-1).astype(jnp.int32)
    nr = n // BR
    t3 = t.reshape(nr, 1, BR)

    row_loss = pl.pallas_call(
        _loss_kernel,
        grid=(nr, NC),
        in_specs=[
            pl.BlockSpec((1, 1, BR), lambda r, c: (r, 0, 0)),
            pl.BlockSpec((BR, BC), lambda r, c: (r, c)),
        ],
        out_specs=pl.BlockSpec((BR, 1), lambda r, c: (r, 0)),
        out_shape=jax.ShapeDtypeStruct((n, 1), jnp.float32),
        scratch_shapes=[
            pltpu.VMEM((BR, 1), jnp.float32),
            pltpu.VMEM((BR, 1), jnp.float32),
            pltpu.VMEM((BR, 1), jnp.float32),
        ],
        compiler_params=pltpu.CompilerParams(
            dimension_semantics=("parallel", "arbitrary")),
    )(t3, p)
    return jnp.sum(row_loss) / n


# speedup vs baseline: 1.0601x; 1.0601x over previous
import jax
import jax.numpy as jnp
from jax.experimental import pallas as pl
from jax.experimental.pallas import tpu as pltpu

V = 32000
PAD = 0
SMOOTHING = 0.1
CONF = 1.0 - SMOOTHING
EPS = SMOOTHING / (V - 2)

BR = 256
BC = 3200
NC = V // BC


def _loss_kernel(t_ref, x_ref, out_ref, m_ref, s_ref, w_ref, p0_ref):
    c = pl.program_id(1)
    x = x_ref[...]

    @pl.when(c == 0)
    def _init():
        m_ref[...] = jnp.full((BR, 1), -jnp.inf, jnp.float32)
        s_ref[...] = jnp.zeros((BR, 1), jnp.float32)
        w_ref[...] = jnp.zeros((BR, 1), jnp.float32)
        p0_ref[...] = x[:, 0:1]

    cmax = jnp.max(x, axis=1, keepdims=True)
    m_old = m_ref[...]
    m_new = jnp.maximum(m_old, cmax)
    alpha = jnp.exp(m_old - m_new)
    s_ref[...] = s_ref[...] * alpha + jnp.sum(
        jnp.exp(x - m_new), axis=1, keepdims=True)
    m_ref[...] = m_new

    t = t_ref[0, 0, :]
    lane = jax.lax.broadcasted_iota(jnp.int32, (BR, BC), 1)
    tloc = t[:, None] - c * BC
    wt = jnp.where(lane == tloc, CONF, EPS)
    w_ref[...] = w_ref[...] + jnp.sum(wt * x, axis=1, keepdims=True)

    @pl.when(c == NC - 1)
    def _finish():
        L = m_ref[...] + jnp.log(s_ref[...])
        loss = L - (w_ref[...] - EPS * p0_ref[...])
        loss = jnp.where(t[:, None] == PAD, 0.0, loss)
        out_ref[...] = loss


def kernel(pred, target):
    n = pred.shape[0] * pred.shape[1]
    p = pred.reshape(n, V)
    t = target.reshape(-1).astype(jnp.int32)
    nr = n // BR
    t3 = t.reshape(nr, 1, BR)

    row_loss = pl.pallas_call(
        _loss_kernel,
        grid=(nr, NC),
        in_specs=[
            pl.BlockSpec((1, 1, BR), lambda r, c: (r, 0, 0)),
            pl.BlockSpec((BR, BC), lambda r, c: (r, c)),
        ],
        out_specs=pl.BlockSpec((BR, 1), lambda r, c: (r, 0)),
        out_shape=jax.ShapeDtypeStruct((n, 1), jnp.float32),
        scratch_shapes=[
            pltpu.VMEM((BR, 1), jnp.float32),
            pltpu.VMEM((BR, 1), jnp.float32),
            pltpu.VMEM((BR, 1), jnp.float32),
            pltpu.VMEM((BR, 1), jnp.float32),
        ],
        compiler_params=pltpu.CompilerParams(
            dimension_semantics=("parallel", "arbitrary")),
    )(t3, p)
    return jnp.sum(row_loss) / n


# speedup vs baseline: 1.2896x; 1.2165x over previous
import jax
import jax.numpy as jnp
from jax.experimental import pallas as pl
from jax.experimental.pallas import tpu as pltpu

V = 32000
PAD = 0
SMOOTHING = 0.1
CONF = 1.0 - SMOOTHING
EPS = SMOOTHING / (V - 2)

BR = 256
BC = 6400
NC = V // BC


def _loss_kernel(t_ref, x_ref, out_ref, m_ref, s_ref, w_ref, p0_ref):
    c = pl.program_id(1)
    x = x_ref[...]

    @pl.when(c == 0)
    def _init():
        m_ref[...] = jnp.full((BR, 1), -jnp.inf, jnp.float32)
        s_ref[...] = jnp.zeros((BR, 1), jnp.float32)
        w_ref[...] = jnp.zeros((BR, 1), jnp.float32)
        p0_ref[...] = x[:, 0:1]

    cmax = jnp.max(x, axis=1, keepdims=True)
    m_old = m_ref[...]
    m_new = jnp.maximum(m_old, cmax)
    alpha = jnp.exp(m_old - m_new)
    s_ref[...] = s_ref[...] * alpha + jnp.sum(
        jnp.exp(x - m_new), axis=1, keepdims=True)
    m_ref[...] = m_new

    t = t_ref[0, 0, :]
    lane = jax.lax.broadcasted_iota(jnp.int32, (BR, BC), 1)
    tloc = t[:, None] - c * BC
    wt = jnp.where(lane == tloc, CONF, EPS)
    w_ref[...] = w_ref[...] + jnp.sum(wt * x, axis=1, keepdims=True)

    @pl.when(c == NC - 1)
    def _finish():
        L = m_ref[...] + jnp.log(s_ref[...])
        loss = L - (w_ref[...] - EPS * p0_ref[...])
        loss = jnp.where(t[:, None] == PAD, 0.0, loss)
        out_ref[...] = loss


def kernel(pred, target):
    n = pred.shape[0] * pred.shape[1]
    p = pred.reshape(n, V)
    t = target.reshape(-1).astype(jnp.int32)
    nr = n // BR
    t3 = t.reshape(nr, 1, BR)

    row_loss = pl.pallas_call(
        _loss_kernel,
        grid=(nr, NC),
        in_specs=[
            pl.BlockSpec((1, 1, BR), lambda r, c: (r, 0, 0)),
            pl.BlockSpec((BR, BC), lambda r, c: (r, c)),
        ],
        out_specs=pl.BlockSpec((BR, 1), lambda r, c: (r, 0)),
        out_shape=jax.ShapeDtypeStruct((n, 1), jnp.float32),
        scratch_shapes=[
            pltpu.VMEM((BR, 1), jnp.float32),
            pltpu.VMEM((BR, 1), jnp.float32),
            pltpu.VMEM((BR, 1), jnp.float32),
            pltpu.VMEM((BR, 1), jnp.float32),
        ],
        compiler_params=pltpu.CompilerParams(
            dimension_semantics=("parallel", "arbitrary")),
    )(t3, p)
    return jnp.sum(row_loss) / n


# speedup vs baseline: 1.3752x; 1.0664x over previous
import jax
import jax.numpy as jnp
from jax.experimental import pallas as pl
from jax.experimental.pallas import tpu as pltpu

V = 32000
PAD = 0
SMOOTHING = 0.1
CONF = 1.0 - SMOOTHING
EPS = SMOOTHING / (V - 2)

BR = 256
BC = 16000
NC = V // BC


def _loss_kernel(t_ref, x_ref, out_ref, m_ref, s_ref, w_ref, p0_ref):
    c = pl.program_id(1)
    x = x_ref[...]

    @pl.when(c == 0)
    def _init():
        m_ref[...] = jnp.full((BR, 1), -jnp.inf, jnp.float32)
        s_ref[...] = jnp.zeros((BR, 1), jnp.float32)
        w_ref[...] = jnp.zeros((BR, 1), jnp.float32)
        p0_ref[...] = x[:, 0:1]

    cmax = jnp.max(x, axis=1, keepdims=True)
    m_old = m_ref[...]
    m_new = jnp.maximum(m_old, cmax)
    alpha = jnp.exp(m_old - m_new)
    s_ref[...] = s_ref[...] * alpha + jnp.sum(
        jnp.exp(x - m_new), axis=1, keepdims=True)
    m_ref[...] = m_new

    t = t_ref[0, 0, :]
    lane = jax.lax.broadcasted_iota(jnp.int32, (BR, BC), 1)
    tloc = t[:, None] - c * BC
    wt = jnp.where(lane == tloc, CONF, EPS)
    w_ref[...] = w_ref[...] + jnp.sum(wt * x, axis=1, keepdims=True)

    @pl.when(c == NC - 1)
    def _finish():
        L = m_ref[...] + jnp.log(s_ref[...])
        loss = L - (w_ref[...] - EPS * p0_ref[...])
        loss = jnp.where(t[:, None] == PAD, 0.0, loss)
        out_ref[...] = loss


def kernel(pred, target):
    n = pred.shape[0] * pred.shape[1]
    p = pred.reshape(n, V)
    t = target.reshape(-1).astype(jnp.int32)
    nr = n // BR
    t3 = t.reshape(nr, 1, BR)

    row_loss = pl.pallas_call(
        _loss_kernel,
        grid=(nr, NC),
        in_specs=[
            pl.BlockSpec((1, 1, BR), lambda r, c: (r, 0, 0)),
            pl.BlockSpec((BR, BC), lambda r, c: (r, c)),
        ],
        out_specs=pl.BlockSpec((BR, 1), lambda r, c: (r, 0)),
        out_shape=jax.ShapeDtypeStruct((n, 1), jnp.float32),
        scratch_shapes=[
            pltpu.VMEM((BR, 1), jnp.float32),
            pltpu.VMEM((BR, 1), jnp.float32),
            pltpu.VMEM((BR, 1), jnp.float32),
            pltpu.VMEM((BR, 1), jnp.float32),
        ],
        compiler_params=pltpu.CompilerParams(
            dimension_semantics=("parallel", "arbitrary")),
    )(t3, p)
    return jnp.sum(row_loss) / n


# speedup vs baseline: 1.5657x; 1.1385x over previous
import jax
import jax.numpy as jnp
from jax.experimental import pallas as pl
from jax.experimental.pallas import tpu as pltpu

V = 32000
PAD = 0
SMOOTHING = 0.1
CONF = 1.0 - SMOOTHING
EPS = SMOOTHING / (V - 2)

BR = 128
BC = 32000
NC = V // BC


def _loss_kernel(t_ref, x_ref, out_ref, m_ref, s_ref, w_ref, p0_ref):
    c = pl.program_id(1)
    x = x_ref[...]

    @pl.when(c == 0)
    def _init():
        m_ref[...] = jnp.full((BR, 1), -jnp.inf, jnp.float32)
        s_ref[...] = jnp.zeros((BR, 1), jnp.float32)
        w_ref[...] = jnp.zeros((BR, 1), jnp.float32)
        p0_ref[...] = x[:, 0:1]

    cmax = jnp.max(x, axis=1, keepdims=True)
    m_old = m_ref[...]
    m_new = jnp.maximum(m_old, cmax)
    alpha = jnp.exp(m_old - m_new)
    s_ref[...] = s_ref[...] * alpha + jnp.sum(
        jnp.exp(x - m_new), axis=1, keepdims=True)
    m_ref[...] = m_new

    t = t_ref[0, 0, :]
    lane = jax.lax.broadcasted_iota(jnp.int32, (BR, BC), 1)
    tloc = t[:, None] - c * BC
    wt = jnp.where(lane == tloc, CONF, EPS)
    w_ref[...] = w_ref[...] + jnp.sum(wt * x, axis=1, keepdims=True)

    @pl.when(c == NC - 1)
    def _finish():
        L = m_ref[...] + jnp.log(s_ref[...])
        loss = L - (w_ref[...] - EPS * p0_ref[...])
        loss = jnp.where(t[:, None] == PAD, 0.0, loss)
        out_ref[...] = loss


def kernel(pred, target):
    n = pred.shape[0] * pred.shape[1]
    p = pred.reshape(n, V)
    t = target.reshape(-1).astype(jnp.int32)
    nr = n // BR
    t3 = t.reshape(nr, 1, BR)

    row_loss = pl.pallas_call(
        _loss_kernel,
        grid=(nr, NC),
        in_specs=[
            pl.BlockSpec((1, 1, BR), lambda r, c: (r, 0, 0)),
            pl.BlockSpec((BR, BC), lambda r, c: (r, c)),
        ],
        out_specs=pl.BlockSpec((BR, 1), lambda r, c: (r, 0)),
        out_shape=jax.ShapeDtypeStruct((n, 1), jnp.float32),
        scratch_shapes=[
            pltpu.VMEM((BR, 1), jnp.float32),
            pltpu.VMEM((BR, 1), jnp.float32),
            pltpu.VMEM((BR, 1), jnp.float32),
            pltpu.VMEM((BR, 1), jnp.float32),
        ],
        compiler_params=pltpu.CompilerParams(
            dimension_semantics=("parallel", "arbitrary")),
    )(t3, p)
    return jnp.sum(row_loss) / n
